# Initial kernel scaffold; baseline (speedup 1.0000x reference)
import jax
import jax.numpy as jnp
from jax.experimental import pallas as pl


def kernel(state_space, w, b, x_idx, s_idx):
    raise NotImplementedError("write your pallas kernel here")


# speedup vs baseline: 103.6997x; 103.6997x over previous
import functools

import jax
import jax.numpy as jnp
from jax import lax
from jax.experimental import pallas as pl
from jax.experimental.pallas import tpu as pltpu
from jax.experimental.pallas import tpu_sc as plsc

N_BATCH = 1024
DIM = 32
N_STATES = 256
RADIUS = 8
TEMP = 2.0
WS = 2 * RADIUS + 1

NUM_WORKERS = 32
ROWS_PER_WORKER = N_BATCH // NUM_WORKERS
GROUPS = ROWS_PER_WORKER // 16


def _tc_tables_body(ls_ref, w_ref, tlin_ref, tlse_ref):
    a = ls_ref[0, 0] / TEMP
    w = w_ref[:]
    jj = lax.broadcasted_iota(jnp.int32, (DIM, WS), 1).astype(jnp.float32) - RADIUS
    t_lin = w * jj * a
    tlin_ref[:] = t_lin
    e = jnp.exp(t_lin)
    kk = lax.broadcasted_iota(jnp.int32, (WS, WS), 0)
    cc = lax.broadcasted_iota(jnp.int32, (WS, WS), 1)
    m = ((kk >= cc - RADIUS) & (kk <= cc + RADIUS)).astype(jnp.float32)
    s = lax.dot_general(e, m, (((1,), (0,)), ((), ())),
                        preferred_element_type=jnp.float32)
    tlse_ref[:] = jnp.log(s)


def _build_tables(ls, w_col):
    return pl.pallas_call(
        _tc_tables_body,
        out_shape=[
            jax.ShapeDtypeStruct((DIM, WS), jnp.float32),
            jax.ShapeDtypeStruct((DIM, WS), jnp.float32),
        ],
    )(ls, w_col)


def _sc_body(xw_hbm, sw_hbm, tlin_hbm, tlse_hbm, out_hbm,
             xv, sv, tlv, tsv, ov):
    wid = lax.axis_index("s") * 2 + lax.axis_index("c")
    pltpu.sync_copy(xw_hbm.at[wid], xv)
    pltpu.sync_copy(sw_hbm.at[wid], sv)
    pltpu.sync_copy(tlin_hbm, tlv)
    pltpu.sync_copy(tlse_hbm, tsv)
    for g in range(GROUPS):
        sl = pl.ds(g * 16, 16)
        acc = jnp.zeros((16,), jnp.float32)
        for i in range(DIM):
            x = xv[i, sl]
            s = sv[i, sl]
            j = jnp.clip(s - x + RADIUS, 0, WS - 1)
            c = RADIUS + jnp.maximum(0, RADIUS - x) \
                - jnp.maximum(0, x - (N_STATES - 1 - RADIUS))
            c = jnp.clip(c, 0, WS - 1)
            base = jnp.full((16,), i * WS, jnp.int32)
            lin = plsc.load_gather(tlv, [base + j])
            lse = plsc.load_gather(tsv, [base + c])
            acc = acc + (lin - lse)
        ov[sl] = acc
    pltpu.sync_copy(ov, out_hbm.at[pl.ds(wid * ROWS_PER_WORKER, ROWS_PER_WORKER)])


@functools.partial(
    pl.kernel,
    mesh=plsc.VectorSubcoreMesh(core_axis_name="c", subcore_axis_name="s"),
    out_type=jax.ShapeDtypeStruct((N_BATCH,), jnp.float32),
    compiler_params=pltpu.CompilerParams(needs_layout_passes=False),
    scratch_types=[
        pltpu.VMEM((DIM, ROWS_PER_WORKER), jnp.int32),
        pltpu.VMEM((DIM, ROWS_PER_WORKER), jnp.int32),
        pltpu.VMEM((DIM * WS,), jnp.float32),
        pltpu.VMEM((DIM * WS,), jnp.float32),
        pltpu.VMEM((ROWS_PER_WORKER,), jnp.float32),
    ],
)
def _sc_gather_reduce(xw, sw, tlin, tlse, out, xv, sv, tlv, tsv, ov):
    _sc_body(xw, sw, tlin, tlse, out, xv, sv, tlv, tsv, ov)


def kernel(state_space, w, b, x_idx, s_idx):
    del b
    ls = (state_space[1] - state_space[0]).reshape(1, 1)
    w_col = w.reshape(DIM, 1).astype(jnp.float32)
    t_lin, t_lse = _build_tables(ls, w_col)
    xw = x_idx.astype(jnp.int32).reshape(NUM_WORKERS, ROWS_PER_WORKER, DIM)
    xw = xw.transpose(0, 2, 1)
    sw = s_idx.astype(jnp.int32).reshape(NUM_WORKERS, ROWS_PER_WORKER, DIM)
    sw = sw.transpose(0, 2, 1)
    return _sc_gather_reduce(xw, sw, t_lin.reshape(DIM * WS),
                             t_lse.reshape(DIM * WS))


# speedup vs baseline: 113.1498x; 1.0911x over previous
import functools

import jax
import jax.numpy as jnp
from jax import lax
from jax.experimental import pallas as pl
from jax.experimental.pallas import tpu as pltpu
from jax.experimental.pallas import tpu_sc as plsc

N_BATCH = 1024
DIM = 32
N_STATES = 256
RADIUS = 8
TEMP = 2.0
WS = 2 * RADIUS + 1

NUM_WORKERS = 32
ROWS_PER_WORKER = N_BATCH // NUM_WORKERS
GROUPS = ROWS_PER_WORKER // 16

LN2 = 0.6931471805599453
_LN_POLY = (0.030449, -0.28382685, 1.11609003, -2.44002976, 3.5140873,
            -1.93675974)


def _vlog(s):
    bits = plsc.bitcast(s, jnp.int32)
    e = (bits >> 23) - 127
    mant = plsc.bitcast((bits & 0x007FFFFF) | 0x3F800000, jnp.float32)
    p = jnp.full((16,), _LN_POLY[0], jnp.float32)
    for coef in _LN_POLY[1:]:
        p = p * mant + coef
    return e.astype(jnp.float32) * LN2 + p


def _sc_body(x_hbm, s_hbm, w_hbm, ls_hbm, out_hbm,
             xv, sv, wv, lsv, tlin, tlse, ov):
    wid = lax.axis_index("s") * 2 + lax.axis_index("c")
    base = wid * ROWS_PER_WORKER
    pltpu.sync_copy(x_hbm.at[pl.ds(base, ROWS_PER_WORKER)], xv)
    pltpu.sync_copy(s_hbm.at[pl.ds(base, ROWS_PER_WORKER)], sv)
    pltpu.sync_copy(w_hbm, wv)
    pltpu.sync_copy(ls_hbm, lsv)

    a = lsv[:] * (1.0 / TEMP)
    for h in range(DIM // 16):
        w_h = wv[pl.ds(h * 16, 16)]
        prefix = []
        run = None
        for k in range(WS):
            arg = w_h * (a * float(k - RADIUS))
            tlin[k, pl.ds(h * 16, 16)] = arg
            run = jnp.exp(arg) if run is None else run + jnp.exp(arg)
            prefix.append(run)
        for c in range(WS):
            hi = min(WS - 1, c + RADIUS)
            ssum = prefix[hi]
            if c - RADIUS - 1 >= 0:
                ssum = ssum - prefix[c - RADIUS - 1]
            tlse[c, pl.ds(h * 16, 16)] = _vlog(ssum)

    iota = lax.iota(jnp.int32, 16)
    for g in range(GROUPS):
        rvec = iota + (g * 16)
        acc = jnp.zeros((16,), jnp.float32)
        for i in range(DIM):
            ifull = jnp.full((16,), i, jnp.int32)
            x = plsc.load_gather(xv, [rvec, ifull])
            s = plsc.load_gather(sv, [rvec, ifull])
            j = jnp.clip(s - x + RADIUS, 0, WS - 1)
            c = RADIUS + jnp.maximum(0, RADIUS - x) \
                - jnp.maximum(0, x - (N_STATES - 1 - RADIUS))
            c = jnp.clip(c, 0, WS - 1)
            lin = plsc.load_gather(tlin, [j, ifull])
            lse = plsc.load_gather(tlse, [c, ifull])
            acc = acc + (lin - lse)
        ov[pl.ds(g * 16, 16)] = acc
    pltpu.sync_copy(ov, out_hbm.at[pl.ds(base, ROWS_PER_WORKER)])


@functools.partial(
    pl.kernel,
    mesh=plsc.VectorSubcoreMesh(core_axis_name="c", subcore_axis_name="s"),
    out_type=jax.ShapeDtypeStruct((N_BATCH,), jnp.float32),
    compiler_params=pltpu.CompilerParams(needs_layout_passes=False),
    scratch_types=[
        pltpu.VMEM((ROWS_PER_WORKER, DIM), jnp.int32),
        pltpu.VMEM((ROWS_PER_WORKER, DIM), jnp.int32),
        pltpu.VMEM((DIM,), jnp.float32),
        pltpu.VMEM((16,), jnp.float32),
        pltpu.VMEM((WS, DIM), jnp.float32),
        pltpu.VMEM((WS, DIM), jnp.float32),
        pltpu.VMEM((ROWS_PER_WORKER,), jnp.float32),
    ],
)
def _sc_fused(x, s, w, ls, out, xv, sv, wv, lsv, tlin, tlse, ov):
    _sc_body(x, s, w, ls, out, xv, sv, wv, lsv, tlin, tlse, ov)


def kernel(state_space, w, b, x_idx, s_idx):
    del b
    ls = jnp.full((16,), state_space[1] - state_space[0], jnp.float32)
    return _sc_fused(x_idx.astype(jnp.int32), s_idx.astype(jnp.int32),
                     w.astype(jnp.float32), ls)


# speedup vs baseline: 124.8660x; 1.1035x over previous
import functools

import jax
import jax.numpy as jnp
from jax import lax
from jax.experimental import pallas as pl
from jax.experimental.pallas import tpu as pltpu
from jax.experimental.pallas import tpu_sc as plsc

N_BATCH = 1024
DIM = 32
N_STATES = 256
RADIUS = 8
TEMP = 2.0
WS = 2 * RADIUS + 1

NUM_WORKERS = 32
ROWS_PER_WORKER = N_BATCH // NUM_WORKERS
GROUPS = ROWS_PER_WORKER // 16

LN2 = 0.6931471805599453
_LN_POLY = (0.030449, -0.28382685, 1.11609003, -2.44002976, 3.5140873,
            -1.93675974)


def _vlog(s):
    bits = plsc.bitcast(s, jnp.int32)
    e = (bits >> 23) - 127
    mant = plsc.bitcast((bits & 0x007FFFFF) | 0x3F800000, jnp.float32)
    p = jnp.full((16,), _LN_POLY[0], jnp.float32)
    for coef in _LN_POLY[1:]:
        p = p * mant + coef
    return e.astype(jnp.float32) * LN2 + p


def _sc_body(x_hbm, s_hbm, w_hbm, ss_hbm, out_hbm,
             xv, sv, wv, ssv, tlin, tlse, ov):
    wid = lax.axis_index("s") * 2 + lax.axis_index("c")
    base = wid * ROWS_PER_WORKER
    pltpu.sync_copy(x_hbm.at[pl.ds(base, ROWS_PER_WORKER)], xv)
    pltpu.sync_copy(s_hbm.at[pl.ds(base, ROWS_PER_WORKER)], sv)
    pltpu.sync_copy(w_hbm, wv)
    pltpu.sync_copy(ss_hbm.at[pl.ds(0, 32)], ssv)

    iota16 = lax.iota(jnp.int32, 16)
    s0 = plsc.load_gather(ssv, [iota16])
    s1 = plsc.load_gather(ssv, [iota16 + 1])
    a = (s1 - s0) * (1.0 / TEMP)
    for h in range(DIM // 16):
        w_h = wv[pl.ds(h * 16, 16)]
        prefix = []
        run = None
        for k in range(WS):
            arg = w_h * (a * float(k - RADIUS))
            tlin[k, pl.ds(h * 16, 16)] = arg
            run = jnp.exp(arg) if run is None else run + jnp.exp(arg)
            prefix.append(run)
        for c in range(WS):
            hi = min(WS - 1, c + RADIUS)
            ssum = prefix[hi]
            if c - RADIUS - 1 >= 0:
                ssum = ssum - prefix[c - RADIUS - 1]
            tlse[c, pl.ds(h * 16, 16)] = _vlog(ssum)

    for g in range(GROUPS):
        rvec = iota16 + (g * 16)
        acc = jnp.zeros((16,), jnp.float32)
        for i in range(DIM):
            ifull = jnp.full((16,), i, jnp.int32)
            x = plsc.load_gather(xv, [rvec, ifull])
            s = plsc.load_gather(sv, [rvec, ifull])
            j = jnp.clip(s - x + RADIUS, 0, WS - 1)
            c = RADIUS + jnp.maximum(0, RADIUS - x) \
                - jnp.maximum(0, x - (N_STATES - 1 - RADIUS))
            c = jnp.clip(c, 0, WS - 1)
            lin = plsc.load_gather(tlin, [j, ifull])
            lse = plsc.load_gather(tlse, [c, ifull])
            acc = acc + (lin - lse)
        ov[pl.ds(g * 16, 16)] = acc
    pltpu.sync_copy(ov, out_hbm.at[pl.ds(base, ROWS_PER_WORKER)])


@functools.partial(
    pl.kernel,
    mesh=plsc.VectorSubcoreMesh(core_axis_name="c", subcore_axis_name="s"),
    out_type=jax.ShapeDtypeStruct((N_BATCH,), jnp.float32),
    compiler_params=pltpu.CompilerParams(needs_layout_passes=False,
                                         skip_device_barrier=True),
    scratch_types=[
        pltpu.VMEM((ROWS_PER_WORKER, DIM), jnp.int32),
        pltpu.VMEM((ROWS_PER_WORKER, DIM), jnp.int32),
        pltpu.VMEM((DIM,), jnp.float32),
        pltpu.VMEM((32,), jnp.float32),
        pltpu.VMEM((WS, DIM), jnp.float32),
        pltpu.VMEM((WS, DIM), jnp.float32),
        pltpu.VMEM((ROWS_PER_WORKER,), jnp.float32),
    ],
)
def _sc_fused(x, s, w, ss, out, xv, sv, wv, ssv, tlin, tlse, ov):
    _sc_body(x, s, w, ss, out, xv, sv, wv, ssv, tlin, tlse, ov)


def kernel(state_space, w, b, x_idx, s_idx):
    del b
    return _sc_fused(x_idx.astype(jnp.int32), s_idx.astype(jnp.int32),
                     w.astype(jnp.float32), state_space.astype(jnp.float32))


# speedup vs baseline: 136.1227x; 1.0902x over previous
import functools

import jax
import jax.numpy as jnp
from jax import lax
from jax.experimental import pallas as pl
from jax.experimental.pallas import tpu as pltpu
from jax.experimental.pallas import tpu_sc as plsc

N_BATCH = 1024
DIM = 32
N_STATES = 256
RADIUS = 8
TEMP = 2.0
WS = 2 * RADIUS + 1

NUM_WORKERS = 32
ROWS_PER_WORKER = N_BATCH // NUM_WORKERS
GROUPS = ROWS_PER_WORKER // 16

LN2 = 0.6931471805599453
_LN_POLY = (0.030449, -0.28382685, 1.11609003, -2.44002976, 3.5140873,
            -1.93675974)


def _vlog(s):
    bits = plsc.bitcast(s, jnp.int32)
    e = (bits >> 23) - 127
    mant = plsc.bitcast((bits & 0x007FFFFF) | 0x3F800000, jnp.float32)
    p = jnp.full((16,), _LN_POLY[0], jnp.float32)
    for coef in _LN_POLY[1:]:
        p = p * mant + coef
    return e.astype(jnp.float32) * LN2 + p


def _sc_body(x_hbm, s_hbm, w_hbm, ss_hbm, out_hbm,
             xv, sv, wv, ssv, tlin, tlse, ov,
             sem_x, sem_s, sem_w, sem_ss):
    wid = lax.axis_index("s") * 2 + lax.axis_index("c")
    base = wid * ROWS_PER_WORKER
    cp_x = pltpu.async_copy(x_hbm.at[pl.ds(base, ROWS_PER_WORKER)], xv, sem_x)
    cp_s = pltpu.async_copy(s_hbm.at[pl.ds(base, ROWS_PER_WORKER)], sv, sem_s)
    cp_w = pltpu.async_copy(w_hbm, wv, sem_w)
    cp_ss = pltpu.async_copy(ss_hbm.at[pl.ds(0, 32)], ssv, sem_ss)
    cp_w.wait()
    cp_ss.wait()

    iota16 = lax.iota(jnp.int32, 16)
    s0 = plsc.load_gather(ssv, [iota16])
    s1 = plsc.load_gather(ssv, [iota16 + 1])
    a = (s1 - s0) * (1.0 / TEMP)
    for h in range(DIM // 16):
        w_h = wv[pl.ds(h * 16, 16)]
        prefix = []
        run = None
        for k in range(WS):
            arg = w_h * (a * float(k - RADIUS))
            tlin[k, pl.ds(h * 16, 16)] = arg
            run = jnp.exp(arg) if run is None else run + jnp.exp(arg)
            prefix.append(run)
        for c in range(WS):
            hi = min(WS - 1, c + RADIUS)
            ssum = prefix[hi]
            if c - RADIUS - 1 >= 0:
                ssum = ssum - prefix[c - RADIUS - 1]
            tlse[c, pl.ds(h * 16, 16)] = _vlog(ssum)

    cp_x.wait()
    cp_s.wait()
    rvecs = tuple(iota16 + (g * 16) for g in range(GROUPS))

    def dim_step(i, accs):
        ifull = jnp.full((16,), i, jnp.int32)
        new_accs = []
        for g in range(GROUPS):
            x = plsc.load_gather(xv, [rvecs[g], ifull])
            s = plsc.load_gather(sv, [rvecs[g], ifull])
            j = jnp.clip(s - x + RADIUS, 0, WS - 1)
            c = RADIUS + jnp.maximum(0, RADIUS - x) \
                - jnp.maximum(0, x - (N_STATES - 1 - RADIUS))
            c = jnp.clip(c, 0, WS - 1)
            lin = plsc.load_gather(tlin, [j, ifull])
            lse = plsc.load_gather(tlse, [c, ifull])
            new_accs.append(accs[g] + (lin - lse))
        return tuple(new_accs)

    accs = lax.fori_loop(0, DIM, dim_step,
                         tuple(jnp.zeros((16,), jnp.float32)
                               for _ in range(GROUPS)))
    for g in range(GROUPS):
        ov[pl.ds(g * 16, 16)] = accs[g]
    pltpu.sync_copy(ov, out_hbm.at[pl.ds(base, ROWS_PER_WORKER)])


@functools.partial(
    pl.kernel,
    mesh=plsc.VectorSubcoreMesh(core_axis_name="c", subcore_axis_name="s"),
    out_type=jax.ShapeDtypeStruct((N_BATCH,), jnp.float32),
    compiler_params=pltpu.CompilerParams(needs_layout_passes=False,
                                         skip_device_barrier=True),
    scratch_types=[
        pltpu.VMEM((ROWS_PER_WORKER, DIM), jnp.int32),
        pltpu.VMEM((ROWS_PER_WORKER, DIM), jnp.int32),
        pltpu.VMEM((DIM,), jnp.float32),
        pltpu.VMEM((32,), jnp.float32),
        pltpu.VMEM((WS, DIM), jnp.float32),
        pltpu.VMEM((WS, DIM), jnp.float32),
        pltpu.VMEM((ROWS_PER_WORKER,), jnp.float32),
        pltpu.SemaphoreType.DMA,
        pltpu.SemaphoreType.DMA,
        pltpu.SemaphoreType.DMA,
        pltpu.SemaphoreType.DMA,
    ],
)
def _sc_fused(x, s, w, ss, out, xv, sv, wv, ssv, tlin, tlse, ov,
              sem_x, sem_s, sem_w, sem_ss):
    _sc_body(x, s, w, ss, out, xv, sv, wv, ssv, tlin, tlse, ov,
             sem_x, sem_s, sem_w, sem_ss)


def kernel(state_space, w, b, x_idx, s_idx):
    del b
    return _sc_fused(x_idx.astype(jnp.int32), s_idx.astype(jnp.int32),
                     w.astype(jnp.float32), state_space.astype(jnp.float32))
